# Initial kernel scaffold; baseline (speedup 1.0000x reference)
import jax
import jax.numpy as jnp
from jax.experimental import pallas as pl


def kernel(x):
    raise NotImplementedError("write your pallas kernel here")


# speedup vs baseline: 3.3147x; 3.3147x over previous
import jax
import jax.numpy as jnp
from jax.experimental import pallas as pl
from jax.experimental.pallas import tpu as pltpu

ROWS = 12
LEN = 65536
SUB = 8
LANES = LEN // SUB
NV = LANES // 128
BINS = 256
EDGES = BINS + 1
DELTA = 255.0 / 256.0
HALF_DELTA = DELTA * 0.5
NACC = 8


def _edge_sums_kernel(x_ref, out_ref, xh_ref):
    xh_ref[...] = x_ref[0] * 0.5

    def body(e, _):
        t = e.astype(jnp.float32) * HALF_DELTA
        accs = [jnp.zeros((SUB, 128), jnp.float32) for _ in range(NACC)]
        for v in range(NV):
            xv = xh_ref[:, v * 128:(v + 1) * 128]
            accs[v % NACC] = accs[v % NACC] + jnp.tanh(xv - t)
        total = accs[0]
        for a in accs[1:]:
            total = total + a
        red = jnp.sum(total, axis=0, keepdims=True)
        out_ref[pl.ds(e, 1), 0, 0, :] = red
        return 0

    jax.lax.fori_loop(0, EDGES, body, 0)


def _finalize_kernel(p_ref, out_ref):
    q = p_ref[0]
    T = jnp.sum(q, axis=0, keepdims=True)
    h = T[:, 0:BINS] - T[:, 1:BINS + 1]
    s = jnp.sum(h, axis=1, keepdims=True)
    out_ref[0] = h / s


def kernel(x):
    x3 = x.reshape(ROWS, SUB, LANES)
    part = pl.pallas_call(
        _edge_sums_kernel,
        grid=(ROWS,),
        in_specs=[pl.BlockSpec((1, SUB, LANES), lambda i: (i, 0, 0))],
        out_specs=pl.BlockSpec((EDGES, 1, 1, 128), lambda i: (0, i, 0, 0)),
        out_shape=jax.ShapeDtypeStruct((EDGES, ROWS, 1, 128), jnp.float32),
        scratch_shapes=[pltpu.VMEM((SUB, LANES), jnp.float32)],
        compiler_params=pltpu.CompilerParams(
            dimension_semantics=("parallel",)),
    )(x3)
    pt = jnp.transpose(part[:, :, 0, :], (1, 2, 0))
    out = pl.pallas_call(
        _finalize_kernel,
        grid=(ROWS,),
        in_specs=[pl.BlockSpec((1, 128, EDGES), lambda i: (i, 0, 0))],
        out_specs=pl.BlockSpec((1, 1, BINS), lambda i: (i, 0, 0)),
        out_shape=jax.ShapeDtypeStruct((ROWS, 1, BINS), jnp.float32),
        compiler_params=pltpu.CompilerParams(
            dimension_semantics=("parallel",)),
    )(pt)
    return out.reshape(4, 3, BINS)


# speedup vs baseline: 4.4444x; 1.3408x over previous
import jax
import jax.numpy as jnp
from jax.experimental import pallas as pl
from jax.experimental.pallas import tpu as pltpu

ROWS = 12
LEN = 65536
SUB = 8
LANES = LEN // SUB
NV = LANES // 128
BINS = 256
EDGES = BINS + 1
DELTA = 255.0 / 256.0
HALF_DELTA = DELTA * 0.5
NACC = 4
EU = 4
EDGES_PAD = 260


def _edge_sums_kernel(x_ref, out_ref, xh_ref):
    xh_ref[...] = x_ref[0] * 0.5

    def body(i, _):
        e0 = i * EU
        ts = [(e0 + u).astype(jnp.float32) * HALF_DELTA for u in range(EU)]
        accs = [[jnp.zeros((SUB, 128), jnp.float32) for _ in range(NACC)]
                for _ in range(EU)]
        for v in range(NV):
            xv = xh_ref[:, v * 128:(v + 1) * 128]
            for u in range(EU):
                accs[u][v % NACC] = accs[u][v % NACC] + jnp.tanh(xv - ts[u])
        for u in range(EU):
            total = accs[u][0]
            for a in accs[u][1:]:
                total = total + a
            red = jnp.sum(total, axis=0, keepdims=True)
            out_ref[pl.ds(e0 + u, 1), 0, 0, :] = red
        return 0

    jax.lax.fori_loop(0, EDGES_PAD // EU, body, 0)


def _finalize_kernel(p_ref, out_ref):
    q = p_ref[0]
    T = jnp.sum(q, axis=0, keepdims=True)
    h = T[:, 0:BINS] - T[:, 1:BINS + 1]
    s = jnp.sum(h, axis=1, keepdims=True)
    out_ref[0] = h / s


def kernel(x):
    x3 = x.reshape(ROWS, SUB, LANES)
    part = pl.pallas_call(
        _edge_sums_kernel,
        grid=(ROWS,),
        in_specs=[pl.BlockSpec((1, SUB, LANES), lambda i: (i, 0, 0))],
        out_specs=pl.BlockSpec((EDGES_PAD, 1, 1, 128), lambda i: (0, i, 0, 0)),
        out_shape=jax.ShapeDtypeStruct((EDGES_PAD, ROWS, 1, 128), jnp.float32),
        scratch_shapes=[pltpu.VMEM((SUB, LANES), jnp.float32)],
        compiler_params=pltpu.CompilerParams(
            dimension_semantics=("arbitrary",)),
    )(x3)
    pt = jnp.transpose(part[:EDGES, :, 0, :], (1, 2, 0))
    out = pl.pallas_call(
        _finalize_kernel,
        grid=(ROWS,),
        in_specs=[pl.BlockSpec((1, 128, EDGES), lambda i: (i, 0, 0))],
        out_specs=pl.BlockSpec((1, 1, BINS), lambda i: (i, 0, 0)),
        out_shape=jax.ShapeDtypeStruct((ROWS, 1, BINS), jnp.float32),
        compiler_params=pltpu.CompilerParams(
            dimension_semantics=("arbitrary",)),
    )(pt)
    return out.reshape(4, 3, BINS)


# speedup vs baseline: 4.6922x; 1.0557x over previous
import jax
import jax.numpy as jnp
from jax.experimental import pallas as pl
from jax.experimental.pallas import tpu as pltpu

ROWS = 12
LEN = 65536
SUB = 8
LANES = LEN // SUB
NV = LANES // 128
BINS = 256
EDGES = BINS + 1
DELTA = 255.0 / 256.0
HALF_DELTA = DELTA * 0.5
NACC = 2
EU = 2
EDGES_PAD = 258
RPB = 3


def _edge_sums_kernel(x_ref, out_ref, xh_ref):
    xh_ref[...] = x_ref[0] * 0.5

    def body(i, _):
        e0 = i * EU
        ts = [(e0 + u).astype(jnp.float32) * HALF_DELTA for u in range(EU)]
        accs = [[[jnp.zeros((SUB, 128), jnp.float32) for _ in range(NACC)]
                 for _ in range(RPB)] for _ in range(EU)]
        for v in range(NV):
            for r in range(RPB):
                xv = xh_ref[r, :, v * 128:(v + 1) * 128]
                for u in range(EU):
                    accs[u][r][v % NACC] = (
                        accs[u][r][v % NACC] + jnp.tanh(xv - ts[u]))
        for u in range(EU):
            for r in range(RPB):
                total = accs[u][r][0]
                for a in accs[u][r][1:]:
                    total = total + a
                red = jnp.sum(total, axis=0, keepdims=True)
                out_ref[pl.ds(e0 + u, 1), 0, r, 0, :] = red
        return 0

    jax.lax.fori_loop(0, EDGES_PAD // EU, body, 0)


def _finalize_kernel(p_ref, out_ref):
    q = p_ref[0]
    T = jnp.sum(q, axis=0, keepdims=True)
    h = T[:, 0:BINS] - T[:, 1:BINS + 1]
    s = jnp.sum(h, axis=1, keepdims=True)
    out_ref[0] = h / s


def kernel(x):
    x4 = x.reshape(ROWS // RPB, RPB, SUB, LANES)
    part = pl.pallas_call(
        _edge_sums_kernel,
        grid=(ROWS // RPB,),
        in_specs=[pl.BlockSpec((1, RPB, SUB, LANES), lambda i: (i, 0, 0, 0))],
        out_specs=pl.BlockSpec((EDGES_PAD, 1, RPB, 1, 128),
                               lambda i: (0, i, 0, 0, 0)),
        out_shape=jax.ShapeDtypeStruct((EDGES_PAD, ROWS // RPB, RPB, 1, 128),
                                       jnp.float32),
        scratch_shapes=[pltpu.VMEM((RPB, SUB, LANES), jnp.float32)],
        compiler_params=pltpu.CompilerParams(
            dimension_semantics=("arbitrary",)),
    )(x4)
    part = part.reshape(EDGES_PAD, ROWS, 128)
    pt = jnp.transpose(part[:EDGES], (1, 2, 0))
    out = pl.pallas_call(
        _finalize_kernel,
        grid=(ROWS,),
        in_specs=[pl.BlockSpec((1, 128, EDGES), lambda i: (i, 0, 0))],
        out_specs=pl.BlockSpec((1, 1, BINS), lambda i: (i, 0, 0)),
        out_shape=jax.ShapeDtypeStruct((ROWS, 1, BINS), jnp.float32),
        compiler_params=pltpu.CompilerParams(
            dimension_semantics=("arbitrary",)),
    )(pt)
    return out.reshape(4, 3, BINS)
